# baseline (speedup 1.0000x reference)
import functools

import jax
import jax.numpy as jnp
from jax import lax
from jax.experimental import pallas as pl
from jax.experimental.pallas import tpu as pltpu
from jax.experimental.pallas import tpu_sc as plsc

N_NODES = 10000
D_FEAT = 128
N_EDGE_ELEMS = 640000
N_ECOLS = N_EDGE_ELEMS // 2

NS = 16
L = 16

A_ROWS_W = 640
A_CHUNK = 128
A_NCHUNKS = A_ROWS_W // A_CHUNK
A_LAST_BASE = N_NODES - A_ROWS_W

B_TILES_W = 157
B_COLS_W = B_TILES_W * 128
B_LAST_TBASE = (N_ECOLS // 128) - B_TILES_W
B_CHUNK_COLS = (79 * 128, 78 * 128)


def _gather_body(x_hbm, perm_hbm, out_x_hbm, idx_v, rows_v, sem_idx,
                 sem_rows, sem_rw):
    t = lax.axis_index("s")
    base = jnp.minimum(t * A_ROWS_W, A_LAST_BASE)

    idx_cps = []
    for j in range(A_NCHUNKS):
        idx_cps.append(pltpu.async_copy(
            perm_hbm.at[pl.ds(base + j * A_CHUNK, A_CHUNK)],
            idx_v.at[j], sem_idx))
    row_cps = []
    for j in range(A_NCHUNKS):
        idx_cps[j].wait()
        row_cps.append(pltpu.async_copy(
            x_hbm.at[idx_v.at[j]],
            rows_v.at[pl.ds(j * A_CHUNK, A_CHUNK)], sem_rows))
    rw_cps = []
    for j in range(A_NCHUNKS):
        row_cps[j].wait()
        rw_cps.append(pltpu.async_copy(
            rows_v.at[pl.ds(j * A_CHUNK, A_CHUNK)],
            out_x_hbm.at[pl.ds(base + j * A_CHUNK, A_CHUNK)], sem_rw))
    for cp in rw_cps:
        cp.wait()


def _edge_body(edge_hbm, perm_hbm, out_e_hbm, perm_v, inv_v, edge_v, eout_v,
               sem_perm, sem_edge, sem_eo):
    t = lax.axis_index("s")

    cp_perm = pltpu.async_copy(perm_hbm, perm_v, sem_perm)
    ecol = jnp.minimum(t * B_TILES_W, B_LAST_TBASE) * 128
    cp_edge = pltpu.async_copy(
        edge_hbm.at[pl.ds(0, 2), pl.ds(ecol, B_COLS_W)], edge_v, sem_edge)

    cp_perm.wait()

    @plsc.parallel_loop(0, N_NODES // L, unroll=8)
    def _inv_loop(i):
        p = perm_v[pl.ds(i * L, L)]
        plsc.store_scatter(inv_v, [p], lax.iota(jnp.int32, L) + i * L)

    cp_edge.wait()

    eout_cps = []
    off = 0
    for ch_cols in B_CHUNK_COLS:
        for r in range(2):
            @plsc.parallel_loop(off // L, (off + ch_cols) // L, unroll=8)
            def _edge_loop(i, r=r):
                e = edge_v[r, pl.ds(i * L, L)]
                eout_v[r, pl.ds(i * L, L)] = plsc.load_gather(inv_v, [e])

        eout_cps.append(pltpu.async_copy(
            eout_v.at[pl.ds(0, 2), pl.ds(off, ch_cols)],
            out_e_hbm.at[pl.ds(0, 2), pl.ds(ecol + off, ch_cols)], sem_eo))
        off += ch_cols

    for cp in eout_cps:
        cp.wait()


@jax.jit
def kernel(x, edge_index, perm):
    edge32 = edge_index.astype(jnp.int32)
    perm32 = perm.astype(jnp.int32)

    run_gather = pl.kernel(
        _gather_body,
        out_type=jax.ShapeDtypeStruct((N_NODES, D_FEAT), jnp.float32),
        mesh=plsc.VectorSubcoreMesh(
            core_axis_name="c", subcore_axis_name="s", num_cores=1),
        compiler_params=pltpu.CompilerParams(needs_layout_passes=False),
        scratch_types=[
            pltpu.VMEM((A_NCHUNKS, A_CHUNK), jnp.int32),
            pltpu.VMEM((A_ROWS_W, D_FEAT), jnp.float32),
            pltpu.SemaphoreType.DMA,
            pltpu.SemaphoreType.DMA,
            pltpu.SemaphoreType.DMA,
        ],
    )
    run_edge = pl.kernel(
        _edge_body,
        out_type=jax.ShapeDtypeStruct((2, N_ECOLS), jnp.int32),
        mesh=plsc.VectorSubcoreMesh(
            core_axis_name="c", subcore_axis_name="s", num_cores=1),
        compiler_params=pltpu.CompilerParams(needs_layout_passes=False),
        scratch_types=[
            pltpu.VMEM((N_NODES,), jnp.int32),
            pltpu.VMEM((N_NODES,), jnp.int32),
            pltpu.VMEM((2, B_COLS_W), jnp.int32),
            pltpu.VMEM((2, B_COLS_W), jnp.int32),
            pltpu.SemaphoreType.DMA,
            pltpu.SemaphoreType.DMA,
            pltpu.SemaphoreType.DMA,
        ],
    )
    new_x = run_gather(x, perm32)
    new_edge = run_edge(edge32, perm32)
    return new_x, new_edge


# speedup vs baseline: 1.2750x; 1.2750x over previous
import functools

import jax
import jax.numpy as jnp
from jax import lax
from jax.experimental import pallas as pl
from jax.experimental.pallas import tpu as pltpu
from jax.experimental.pallas import tpu_sc as plsc

N_NODES = 10000
D_FEAT = 128
N_EDGE_ELEMS = 640000

NC = 2
NS = 16
NW = NC * NS
L = 16

ROWS_PER_W = 320
X_CHUNK = 80
N_XCHUNKS = ROWS_PER_W // X_CHUNK
EDGES_PER_W = N_EDGE_ELEMS // NW


N_ECOLS = N_EDGE_ELEMS // 2
E_TILES_W = 79
E_COLS_W = E_TILES_W * 128
E_LAST_TBASE = (N_ECOLS // 128) - E_TILES_W
E_CHUNK_COLS = (40 * 128, 39 * 128)


def _sc_body(x_hbm, edge_hbm, perm_hbm, out_x_hbm, out_e_hbm,
             perm_v, inv_v, idx_v, rows_v, edge_v, eout_v,
             sem_perm, sem_idx, sem_edge, sem_rows, sem_eo, sem_rw):
    c = lax.axis_index("c")
    s = lax.axis_index("s")
    wid = s * NC + c

    xbase = jnp.minimum(wid * ROWS_PER_W, N_NODES - ROWS_PER_W)
    idx_cps = []
    for j in range(N_XCHUNKS):
        idx_cps.append(pltpu.async_copy(
            perm_hbm.at[pl.ds(xbase + j * X_CHUNK, X_CHUNK)],
            idx_v.at[j], sem_idx))

    ecol = jnp.minimum(wid * E_TILES_W, E_LAST_TBASE) * 128
    cp_edge = pltpu.async_copy(
        edge_hbm.at[pl.ds(0, 2), pl.ds(ecol, E_COLS_W)], edge_v, sem_edge)

    cp_perm = pltpu.async_copy(perm_hbm, perm_v, sem_perm)

    row_cps = []
    for j in range(N_XCHUNKS):
        idx_cps[j].wait()
        row_cps.append(pltpu.async_copy(
            x_hbm.at[idx_v.at[j]],
            rows_v.at[pl.ds(j * X_CHUNK, X_CHUNK)], sem_rows))

    cp_perm.wait()

    @plsc.parallel_loop(0, N_NODES // L, unroll=8)
    def _inv_loop(i):
        p = perm_v[pl.ds(i * L, L)]
        plsc.store_scatter(inv_v, [p], lax.iota(jnp.int32, L) + i * L)

    cp_edge.wait()

    eout_cps = []
    rw_cps = []
    off = 0
    for n_chunk, ch_cols in enumerate(E_CHUNK_COLS):
        for r in range(2):
            @plsc.parallel_loop(off // L, (off + ch_cols) // L, unroll=8)
            def _edge_loop(i, r=r):
                e = edge_v[r, pl.ds(i * L, L)]
                eout_v[r, pl.ds(i * L, L)] = plsc.load_gather(inv_v, [e])

        eout_cps.append(pltpu.async_copy(
            eout_v.at[pl.ds(0, 2), pl.ds(off, ch_cols)],
            out_e_hbm.at[pl.ds(0, 2), pl.ds(ecol + off, ch_cols)], sem_eo))
        off += ch_cols

        if n_chunk == 0:
            for j in range(N_XCHUNKS):
                row_cps[j].wait()
                rw_cps.append(pltpu.async_copy(
                    rows_v.at[pl.ds(j * X_CHUNK, X_CHUNK)],
                    out_x_hbm.at[pl.ds(xbase + j * X_CHUNK, X_CHUNK)],
                    sem_rw))

    for cp in eout_cps:
        cp.wait()
    for cp in rw_cps:
        cp.wait()


@jax.jit
def kernel(x, edge_index, perm):
    edge32 = edge_index.astype(jnp.int32)
    perm32 = perm.astype(jnp.int32)

    run = pl.kernel(
        _sc_body,
        out_type=(
            jax.ShapeDtypeStruct((N_NODES, D_FEAT), jnp.float32),
            jax.ShapeDtypeStruct((2, N_EDGE_ELEMS // 2), jnp.int32),
        ),
        mesh=plsc.VectorSubcoreMesh(
            core_axis_name="c", subcore_axis_name="s"),
        compiler_params=pltpu.CompilerParams(needs_layout_passes=False),
        scratch_types=[
            pltpu.VMEM((N_NODES,), jnp.int32),
            pltpu.VMEM((N_NODES,), jnp.int32),
            pltpu.VMEM((N_XCHUNKS, X_CHUNK), jnp.int32),
            pltpu.VMEM((ROWS_PER_W, D_FEAT), jnp.float32),
            pltpu.VMEM((2, E_COLS_W), jnp.int32),
            pltpu.VMEM((2, E_COLS_W), jnp.int32),
            pltpu.SemaphoreType.DMA,
            pltpu.SemaphoreType.DMA,
            pltpu.SemaphoreType.DMA,
            pltpu.SemaphoreType.DMA,
            pltpu.SemaphoreType.DMA,
            pltpu.SemaphoreType.DMA,
        ],
    )
    new_x, new_edge = run(x, edge32, perm32)
    return new_x, new_edge
